# baseline (speedup 1.0000x reference)
import jax
import jax.numpy as jnp
from jax import lax
from jax.experimental import pallas as pl
from jax.experimental.pallas import tpu as pltpu
from jax.experimental.pallas import tpu_sc as plsc

N = 10000
E = 320000
G = 128
NODE_DIM = 128
EDGE_DIM = 16
L = 3

_NC = 2
_NS = 16
_NW = _NC * _NS
CH = 512
NCHUNK = E // CH
NJ = (NCHUNK + _NW - 1) // _NW
ROWS_PER_SUB = N // _NS
N8 = N // 8
E8 = E // 8


def _mmb_body(x_ref, w_ref, b_ref, o_ref):
    o_ref[...] = jnp.dot(x_ref[...], w_ref[...],
                         preferred_element_type=jnp.float32) + b_ref[...]


def _matmul_bias(x, w, b, block_rows):
    rows = x.shape[0]
    grid = rows // block_rows
    return pl.pallas_call(
        _mmb_body,
        grid=(grid,),
        in_specs=[
            pl.BlockSpec((block_rows, x.shape[1]), lambda i: (i, 0)),
            pl.BlockSpec(w.shape, lambda i: (0, 0)),
            pl.BlockSpec((1, w.shape[1]), lambda i: (0, 0)),
        ],
        out_specs=pl.BlockSpec((block_rows, w.shape[1]), lambda i: (i, 0)),
        out_shape=jax.ShapeDtypeStruct((rows, w.shape[1]), jnp.float32),
    )(x, w, b.reshape(1, -1))


_BE = 2000
_NEB = E8 // _BE


def _embedproj_t0_body(nf8_ref, wat_ref, ba_ref, ws_ref, wd_ref, bs_ref,
                       bd_ref, ef_ref, w0_ref, b0_ref,
                       h8_ref, ps_ref, pd_ref, t_ref):
    i = pl.program_id(0)

    @pl.when(i == 0)
    def _node():
        nf8 = nf8_ref[...]
        h8_ref[...] = jnp.dot(nf8, wat_ref[...],
                              preferred_element_type=jnp.float32) + ba_ref[...]
        ps_ref[...] = jnp.dot(nf8, ws_ref[...],
                              preferred_element_type=jnp.float32) + bs_ref[...]
        pd_ref[...] = jnp.dot(nf8, wd_ref[...],
                              preferred_element_type=jnp.float32) + bd_ref[...]

    @pl.when(i > 0)
    def _edge():
        t_ref[...] = jnp.dot(ef_ref[...], w0_ref[...],
                             preferred_element_type=jnp.float32) + b0_ref[...]


def _embedproj_t0(nf8, wat_k, ba8, ws_c, wd_c, bs0, bd0, ef8, w0c, b0c):
    zz = lambda i: (0, 0)
    eb = lambda i: (jnp.maximum(i - 1, 0), 0)
    return pl.pallas_call(
        _embedproj_t0_body,
        grid=(1 + _NEB,),
        in_specs=[
            pl.BlockSpec((N8, 8 * NODE_DIM), zz),
            pl.BlockSpec((8 * NODE_DIM, 8 * NODE_DIM), zz),
            pl.BlockSpec((1, 8 * NODE_DIM), zz),
            pl.BlockSpec((8 * NODE_DIM, 8 * EDGE_DIM), zz),
            pl.BlockSpec((8 * NODE_DIM, 8 * EDGE_DIM), zz),
            pl.BlockSpec((1, 8 * EDGE_DIM), zz),
            pl.BlockSpec((1, 8 * EDGE_DIM), zz),
            pl.BlockSpec((_BE, 8 * EDGE_DIM), eb),
            pl.BlockSpec((8 * EDGE_DIM, 8 * EDGE_DIM), zz),
            pl.BlockSpec((1, 8 * EDGE_DIM), zz),
        ],
        out_specs=[
            pl.BlockSpec((N8, 8 * NODE_DIM), zz),
            pl.BlockSpec((N8, 8 * EDGE_DIM), zz),
            pl.BlockSpec((N8, 8 * EDGE_DIM), zz),
            pl.BlockSpec((_BE, 8 * EDGE_DIM), eb),
        ],
        out_shape=[
            jax.ShapeDtypeStruct((N8, 8 * NODE_DIM), jnp.float32),
            jax.ShapeDtypeStruct((N8, 8 * EDGE_DIM), jnp.float32),
            jax.ShapeDtypeStruct((N8, 8 * EDGE_DIM), jnp.float32),
            jax.ShapeDtypeStruct((E8, 8 * EDGE_DIM), jnp.float32),
        ],
    )(nf8, wat_k, ba8.reshape(1, -1), ws_c, wd_c,
      bs0.reshape(1, -1), bd0.reshape(1, -1), ef8, w0c, b0c.reshape(1, -1))


def _updproj_tmsg_body(h8_ref, agg8_ref, wh_ref, wa_ref, b_ref, ws_ref,
                       wd_ref, e8_ref, we_ref, be_ref,
                       o_ref, ps_ref, pd_ref, t_ref):
    i = pl.program_id(0)

    @pl.when(i == 0)
    def _node():
        a = agg8_ref[0] + agg8_ref[1]
        acc = jnp.dot(h8_ref[...], wh_ref[...],
                      preferred_element_type=jnp.float32)
        acc = acc + jnp.dot(a, wa_ref[...], preferred_element_type=jnp.float32)
        h8 = jnp.maximum(acc + b_ref[...], 0.0)
        o_ref[...] = h8
        ps_ref[...] = jnp.dot(h8, ws_ref[...],
                              preferred_element_type=jnp.float32)
        pd_ref[...] = jnp.dot(h8, wd_ref[...],
                              preferred_element_type=jnp.float32)

    @pl.when(i > 0)
    def _edge():
        t_ref[...] = jnp.dot(e8_ref[...], we_ref[...],
                             preferred_element_type=jnp.float32) + be_ref[...]


def _updproj_tmsg(h8, agg8, wh_k, wa_k, b8, ws_k, wd_k, e8, we_k, be8):
    zz = lambda i: (0, 0)
    eb = lambda i: (jnp.maximum(i - 1, 0), 0)
    return pl.pallas_call(
        _updproj_tmsg_body,
        grid=(1 + _NEB,),
        in_specs=[
            pl.BlockSpec((N8, 8 * NODE_DIM), zz),
            pl.BlockSpec((2, N8, 8 * EDGE_DIM), lambda i: (0, 0, 0)),
            pl.BlockSpec((8 * NODE_DIM, 8 * NODE_DIM), zz),
            pl.BlockSpec((8 * EDGE_DIM, 8 * NODE_DIM), zz),
            pl.BlockSpec((1, 8 * NODE_DIM), zz),
            pl.BlockSpec((8 * NODE_DIM, 8 * EDGE_DIM), zz),
            pl.BlockSpec((8 * NODE_DIM, 8 * EDGE_DIM), zz),
            pl.BlockSpec((_BE, 8 * EDGE_DIM), eb),
            pl.BlockSpec((8 * EDGE_DIM, 8 * EDGE_DIM), zz),
            pl.BlockSpec((1, 8 * EDGE_DIM), zz),
        ],
        out_specs=[
            pl.BlockSpec((N8, 8 * NODE_DIM), zz),
            pl.BlockSpec((N8, 8 * EDGE_DIM), zz),
            pl.BlockSpec((N8, 8 * EDGE_DIM), zz),
            pl.BlockSpec((_BE, 8 * EDGE_DIM), eb),
        ],
        out_shape=[
            jax.ShapeDtypeStruct((N8, 8 * NODE_DIM), jnp.float32),
            jax.ShapeDtypeStruct((N8, 8 * EDGE_DIM), jnp.float32),
            jax.ShapeDtypeStruct((N8, 8 * EDGE_DIM), jnp.float32),
            jax.ShapeDtypeStruct((E8, 8 * EDGE_DIM), jnp.float32),
        ],
    )(h8, agg8, wh_k, wa_k, b8.reshape(1, -1), ws_k, wd_k,
      e8, we_k, be8.reshape(1, -1))


def _node_update_body(h8_ref, agg8_ref, wh_ref, wa_ref, b_ref, o_ref):
    a = agg8_ref[0] + agg8_ref[1]
    acc = jnp.dot(h8_ref[...], wh_ref[...], preferred_element_type=jnp.float32)
    acc = acc + jnp.dot(a, wa_ref[...], preferred_element_type=jnp.float32)
    o_ref[...] = jnp.maximum(acc + b_ref[...], 0.0)


def _node_update(h8, agg8, wh_k, wa_k, b8):
    return pl.pallas_call(
        _node_update_body,
        grid=(1,),
        in_specs=[
            pl.BlockSpec((N8, 8 * NODE_DIM), lambda i: (0, 0)),
            pl.BlockSpec((2, N8, 8 * EDGE_DIM), lambda i: (0, 0, 0)),
            pl.BlockSpec((8 * NODE_DIM, 8 * NODE_DIM), lambda i: (0, 0)),
            pl.BlockSpec((8 * EDGE_DIM, 8 * NODE_DIM), lambda i: (0, 0)),
            pl.BlockSpec((1, 8 * NODE_DIM), lambda i: (0, 0)),
        ],
        out_specs=pl.BlockSpec((N8, 8 * NODE_DIM), lambda i: (0, 0)),
        out_shape=jax.ShapeDtypeStruct((N8, 8 * NODE_DIM), jnp.float32),
    )(h8, agg8, wh_k, wa_k, b8.reshape(1, -1))


def _readout_body(gi_ref, h_ref, wg_ref, bg_ref, wo_ref, o_ref, pooled_ref):
    i = pl.program_id(0)
    ng = pl.num_programs(0)

    @pl.when(i == 0)
    def _init():
        pooled_ref[...] = jnp.zeros_like(pooled_ref)

    gi = gi_ref[...]
    iota = lax.broadcasted_iota(jnp.int32, (gi.shape[0], G), 1).astype(jnp.float32)
    onehot = (gi == iota).astype(jnp.float32)
    contrib = lax.dot_general(onehot, h_ref[...],
                              (((0,), (0,)), ((), ())),
                              preferred_element_type=jnp.float32)
    pooled_ref[...] += contrib

    @pl.when(i == ng - 1)
    def _fin():
        g = jnp.maximum(
            jnp.dot(pooled_ref[...], wg_ref[...],
                    preferred_element_type=jnp.float32) + bg_ref[...], 0.0)
        o_ref[...] = jnp.dot(g, wo_ref[...],
                             preferred_element_type=jnp.float32)


def _readout(gi_f, h, wg, bg, wo, block_rows=1000):
    grid = N // block_rows
    return pl.pallas_call(
        _readout_body,
        grid=(grid,),
        in_specs=[
            pl.BlockSpec((block_rows, 1), lambda i: (i, 0)),
            pl.BlockSpec((block_rows, NODE_DIM), lambda i: (i, 0)),
            pl.BlockSpec((NODE_DIM, G), lambda i: (0, 0)),
            pl.BlockSpec((1, G), lambda i: (0, 0)),
            pl.BlockSpec((G, 256), lambda i: (0, 0)),
        ],
        out_specs=pl.BlockSpec((G, 256), lambda i: (0, 0)),
        out_shape=jax.ShapeDtypeStruct((G, 256), jnp.float32),
        scratch_shapes=[pltpu.VMEM((G, G), jnp.float32)],
    )(gi_f, h, wg, bg.reshape(1, -1), wo)


def _make_edge_body(write_eout):
    def body(ei_h, ps_h, pd_h, t_h, *rest):
        if write_eout:
            eout_h, agg_h = rest[0], rest[1]
            scratch = rest[2:]
        else:
            agg_h = rest[0]
            scratch = rest[1:]
        (sidxA, didxA, psvA, pdvA, tvA,
         sidxB, didxB, psvB, pdvB, tvB,
         ev, obuf, agg_sh, semA, semB) = scratch
        c = lax.axis_index("c")
        s = lax.axis_index("s")
        wid = s * _NC + c
        CR = CH // 8

        def fire(j, sidx, didx, psv, pdv, tv, sem):
            cid = j * _NW + wid

            @pl.when(cid < NCHUNK)
            def _():
                pltpu.sync_copy(ei_h.at[0, pl.ds(cid * CH, CH)], sidx)
                pltpu.sync_copy(ei_h.at[1, pl.ds(cid * CH, CH)], didx)
                pltpu.async_copy(ps_h.at[sidx], psv, sem)
                pltpu.async_copy(pd_h.at[didx], pdv, sem)
                pltpu.async_copy(t_h.at[pl.ds(cid * CH, CH), :], tv, sem)

        def drain(j, psv, pdv, tv, sem):
            cid = j * _NW + wid

            @pl.when(cid < NCHUNK)
            def _():
                pltpu.make_async_copy(ps_h.at[pl.ds(0, CH), :],
                                      psv, sem).wait()
                pltpu.make_async_copy(ps_h.at[pl.ds(0, CH), :],
                                      pdv, sem).wait()
                pltpu.make_async_copy(t_h.at[pl.ds(0, CH), :], tv, sem).wait()

        def process(j, didx, psv, pdv, tv):
            cid = j * _NW + wid

            @pl.when(cid < NCHUNK)
            def _():
                def _row(i, carry):
                    tv[i, :] = jnp.maximum(psv[i, :] + pdv[i, :] + tv[i, :],
                                           0.0)
                    return carry
                lax.fori_loop(0, CH, _row, 0)
                if write_eout:
                    pltpu.sync_copy(tv, eout_h.at[pl.ds(cid * CH, CH), :])
                pltpu.sync_copy(tv, agg_sh.at[didx], add=True)

        fire(0, sidxA, didxA, psvA, pdvA, tvA, semA)

        def _zrow(i, carry):
            obuf[i, :] = jnp.zeros((16,), jnp.float32)
            return carry
        lax.fori_loop(0, ROWS_PER_SUB, _zrow, 0)
        pltpu.sync_copy(obuf,
                        agg_sh.at[pl.ds(s * ROWS_PER_SUB, ROWS_PER_SUB), :])
        plsc.subcore_barrier()

        def loop(j2, carry):
            jA = 2 * j2
            jB = jA + 1
            fire(jB, sidxB, didxB, psvB, pdvB, tvB, semB)
            drain(jA, psvA, pdvA, tvA, semA)
            process(jA, didxA, psvA, pdvA, tvA)
            fire(jA + 2, sidxA, didxA, psvA, pdvA, tvA, semA)
            drain(jB, psvB, pdvB, tvB, semB)
            process(jB, didxB, psvB, pdvB, tvB)
            return carry
        lax.fori_loop(0, NJ // 2, loop, 0)

        plsc.subcore_barrier()
        pltpu.sync_copy(agg_sh.at[pl.ds(s * ROWS_PER_SUB, ROWS_PER_SUB), :],
                        obuf)
        pltpu.sync_copy(obuf,
                        agg_h.at[c, pl.ds(s * ROWS_PER_SUB, ROWS_PER_SUB), :])
    return body


def _edge_stage(ei2, ps, pd, t, write_eout=True):
    mesh = plsc.VectorSubcoreMesh(core_axis_name="c", subcore_axis_name="s",
                                  num_cores=_NC, num_subcores=_NS)
    agg_ty = jax.ShapeDtypeStruct((_NC, N, EDGE_DIM), jnp.float32)
    if write_eout:
        out_type = (jax.ShapeDtypeStruct((E, EDGE_DIM), jnp.float32),
                    agg_ty)
    else:
        out_type = (agg_ty,)
    f = pl.kernel(
        _make_edge_body(write_eout),
        out_type=out_type,
        mesh=mesh,
        compiler_params=pltpu.CompilerParams(use_tc_tiling_on_sc=False),
        scratch_types=[
            pltpu.VMEM((CH,), jnp.int32),
            pltpu.VMEM((CH,), jnp.int32),
            pltpu.VMEM((CH, EDGE_DIM), jnp.float32),
            pltpu.VMEM((CH, EDGE_DIM), jnp.float32),
            pltpu.VMEM((CH, EDGE_DIM), jnp.float32),
            pltpu.VMEM((CH,), jnp.int32),
            pltpu.VMEM((CH,), jnp.int32),
            pltpu.VMEM((CH, EDGE_DIM), jnp.float32),
            pltpu.VMEM((CH, EDGE_DIM), jnp.float32),
            pltpu.VMEM((CH, EDGE_DIM), jnp.float32),
            pltpu.VMEM((CH, EDGE_DIM), jnp.float32),
            pltpu.VMEM((ROWS_PER_SUB, EDGE_DIM), jnp.float32),
            pltpu.VMEM_SHARED((N, EDGE_DIM), jnp.float32),
            pltpu.SemaphoreType.DMA,
            pltpu.SemaphoreType.DMA,
        ],
    )
    return f(ei2, ps, pd, t)


def kernel(node_features, edge_features, edge_index, graph_index,
           W_atom, b_atom, W_bond, b_bond, W_edge, b_edge,
           W_node, b_node, W_graph, b_graph, W_out):
    ei2 = edge_index.astype(jnp.int32)
    eye8 = jnp.eye(8, dtype=jnp.float32)

    def ekron(w):
        return jnp.kron(eye8, w)

    ws_k = [ekron(W_edge[l, :NODE_DIM, :]) for l in range(L)]
    wd_k = [ekron(W_edge[l, NODE_DIM:2 * NODE_DIM, :]) for l in range(L)]
    we8 = [ekron(W_edge[l, 2 * NODE_DIM:, :]) for l in range(L)]
    be8 = [jnp.tile(b_edge[l], 8) for l in range(L)]
    wh_k = [ekron(W_node[l, :NODE_DIM, :]) for l in range(L)]
    wa_k = [ekron(W_node[l, NODE_DIM:, :]) for l in range(L)]
    bn8 = [jnp.tile(b_node[l], 8) for l in range(L)]

    nf8 = node_features.reshape(N8, 8 * NODE_DIM)
    wat_k = ekron(W_atom)
    ba8 = jnp.tile(b_atom, 8)
    ws0c = ekron(W_atom @ W_edge[0, :NODE_DIM, :])
    wd0c = ekron(W_atom @ W_edge[0, NODE_DIM:2 * NODE_DIM, :])
    bs0 = jnp.tile(b_atom @ W_edge[0, :NODE_DIM, :], 8)
    bd0 = jnp.tile(b_atom @ W_edge[0, NODE_DIM:2 * NODE_DIM, :], 8)
    ef8 = edge_features.reshape(E8, 8 * EDGE_DIM)
    wb8 = ekron(W_bond)
    bb8 = jnp.tile(b_bond, 8)
    w0c = wb8 @ we8[0]
    b0c = bb8 @ we8[0] + be8[0]
    h8, ps8, pd8, t8 = _embedproj_t0(nf8, wat_k, ba8, ws0c, wd0c, bs0, bd0,
                                     ef8, w0c, b0c)

    for l in range(L):
        last = l == L - 1
        outs = _edge_stage(ei2,
                           ps8.reshape(N, EDGE_DIM),
                           pd8.reshape(N, EDGE_DIM),
                           t8.reshape(E, EDGE_DIM),
                           write_eout=not last)
        if last:
            (aggp,) = outs
            h8 = _node_update(h8, aggp.reshape(_NC, N8, 8 * EDGE_DIM),
                              wh_k[l], wa_k[l], bn8[l])
        else:
            e_new, aggp = outs
            h8, ps8, pd8, t8 = _updproj_tmsg(
                h8, aggp.reshape(_NC, N8, 8 * EDGE_DIM),
                wh_k[l], wa_k[l], bn8[l],
                ws_k[l + 1], wd_k[l + 1],
                e_new.reshape(E8, 8 * EDGE_DIM), we8[l + 1], be8[l + 1])

    gi_f = graph_index.astype(jnp.float32).reshape(N, 1)
    return _readout(gi_f, h8.reshape(N, NODE_DIM), W_graph, b_graph, W_out)


# speedup vs baseline: 1.0252x; 1.0252x over previous
import jax
import jax.numpy as jnp
from jax import lax
from jax.experimental import pallas as pl
from jax.experimental.pallas import tpu as pltpu
from jax.experimental.pallas import tpu_sc as plsc

N = 10000
E = 320000
G = 128
NODE_DIM = 128
EDGE_DIM = 16
L = 3

_NC = 2
_NS = 16
_NW = _NC * _NS
CH = 640
NCHUNK = E // CH
NJ = (NCHUNK + _NW - 1) // _NW
ROWS_PER_SUB = N // _NS
N8 = N // 8
E8 = E // 8


def _mmb_body(x_ref, w_ref, b_ref, o_ref):
    o_ref[...] = jnp.dot(x_ref[...], w_ref[...],
                         preferred_element_type=jnp.float32) + b_ref[...]


def _matmul_bias(x, w, b, block_rows):
    rows = x.shape[0]
    grid = rows // block_rows
    return pl.pallas_call(
        _mmb_body,
        grid=(grid,),
        in_specs=[
            pl.BlockSpec((block_rows, x.shape[1]), lambda i: (i, 0)),
            pl.BlockSpec(w.shape, lambda i: (0, 0)),
            pl.BlockSpec((1, w.shape[1]), lambda i: (0, 0)),
        ],
        out_specs=pl.BlockSpec((block_rows, w.shape[1]), lambda i: (i, 0)),
        out_shape=jax.ShapeDtypeStruct((rows, w.shape[1]), jnp.float32),
    )(x, w, b.reshape(1, -1))


_BE = 2000
_NEB = E8 // _BE


def _embedproj_t0_body(nf8_ref, wat_ref, ba_ref, ws_ref, wd_ref, bs_ref,
                       bd_ref, ef_ref, w0_ref, b0_ref,
                       h8_ref, ps_ref, pd_ref, t_ref):
    i = pl.program_id(0)

    @pl.when(i == 0)
    def _node():
        nf8 = nf8_ref[...]
        h8_ref[...] = jnp.dot(nf8, wat_ref[...],
                              preferred_element_type=jnp.float32) + ba_ref[...]
        ps_ref[...] = jnp.dot(nf8, ws_ref[...],
                              preferred_element_type=jnp.float32) + bs_ref[...]
        pd_ref[...] = jnp.dot(nf8, wd_ref[...],
                              preferred_element_type=jnp.float32) + bd_ref[...]

    @pl.when(i > 0)
    def _edge():
        t_ref[...] = jnp.dot(ef_ref[...], w0_ref[...],
                             preferred_element_type=jnp.float32) + b0_ref[...]


def _embedproj_t0(nf8, wat_k, ba8, ws_c, wd_c, bs0, bd0, ef8, w0c, b0c):
    zz = lambda i: (0, 0)
    eb = lambda i: (jnp.maximum(i - 1, 0), 0)
    return pl.pallas_call(
        _embedproj_t0_body,
        grid=(1 + _NEB,),
        in_specs=[
            pl.BlockSpec((N8, 8 * NODE_DIM), zz),
            pl.BlockSpec((8 * NODE_DIM, 8 * NODE_DIM), zz),
            pl.BlockSpec((1, 8 * NODE_DIM), zz),
            pl.BlockSpec((8 * NODE_DIM, 8 * EDGE_DIM), zz),
            pl.BlockSpec((8 * NODE_DIM, 8 * EDGE_DIM), zz),
            pl.BlockSpec((1, 8 * EDGE_DIM), zz),
            pl.BlockSpec((1, 8 * EDGE_DIM), zz),
            pl.BlockSpec((_BE, 8 * EDGE_DIM), eb),
            pl.BlockSpec((8 * EDGE_DIM, 8 * EDGE_DIM), zz),
            pl.BlockSpec((1, 8 * EDGE_DIM), zz),
        ],
        out_specs=[
            pl.BlockSpec((N8, 8 * NODE_DIM), zz),
            pl.BlockSpec((N8, 8 * EDGE_DIM), zz),
            pl.BlockSpec((N8, 8 * EDGE_DIM), zz),
            pl.BlockSpec((_BE, 8 * EDGE_DIM), eb),
        ],
        out_shape=[
            jax.ShapeDtypeStruct((N8, 8 * NODE_DIM), jnp.float32),
            jax.ShapeDtypeStruct((N8, 8 * EDGE_DIM), jnp.float32),
            jax.ShapeDtypeStruct((N8, 8 * EDGE_DIM), jnp.float32),
            jax.ShapeDtypeStruct((E8, 8 * EDGE_DIM), jnp.float32),
        ],
    )(nf8, wat_k, ba8.reshape(1, -1), ws_c, wd_c,
      bs0.reshape(1, -1), bd0.reshape(1, -1), ef8, w0c, b0c.reshape(1, -1))


def _updproj_tmsg_body(h8_ref, agg8_ref, wh_ref, wa_ref, b_ref, ws_ref,
                       wd_ref, e8_ref, we_ref, be_ref,
                       o_ref, ps_ref, pd_ref, t_ref):
    i = pl.program_id(0)

    @pl.when(i == 0)
    def _node():
        a = agg8_ref[0] + agg8_ref[1]
        acc = jnp.dot(h8_ref[...], wh_ref[...],
                      preferred_element_type=jnp.float32)
        acc = acc + jnp.dot(a, wa_ref[...], preferred_element_type=jnp.float32)
        h8 = jnp.maximum(acc + b_ref[...], 0.0)
        o_ref[...] = h8
        ps_ref[...] = jnp.dot(h8, ws_ref[...],
                              preferred_element_type=jnp.float32)
        pd_ref[...] = jnp.dot(h8, wd_ref[...],
                              preferred_element_type=jnp.float32)

    @pl.when(i > 0)
    def _edge():
        t_ref[...] = jnp.dot(e8_ref[...], we_ref[...],
                             preferred_element_type=jnp.float32) + be_ref[...]


def _updproj_tmsg(h8, agg8, wh_k, wa_k, b8, ws_k, wd_k, e8, we_k, be8):
    zz = lambda i: (0, 0)
    eb = lambda i: (jnp.maximum(i - 1, 0), 0)
    return pl.pallas_call(
        _updproj_tmsg_body,
        grid=(1 + _NEB,),
        in_specs=[
            pl.BlockSpec((N8, 8 * NODE_DIM), zz),
            pl.BlockSpec((2, N8, 8 * EDGE_DIM), lambda i: (0, 0, 0)),
            pl.BlockSpec((8 * NODE_DIM, 8 * NODE_DIM), zz),
            pl.BlockSpec((8 * EDGE_DIM, 8 * NODE_DIM), zz),
            pl.BlockSpec((1, 8 * NODE_DIM), zz),
            pl.BlockSpec((8 * NODE_DIM, 8 * EDGE_DIM), zz),
            pl.BlockSpec((8 * NODE_DIM, 8 * EDGE_DIM), zz),
            pl.BlockSpec((_BE, 8 * EDGE_DIM), eb),
            pl.BlockSpec((8 * EDGE_DIM, 8 * EDGE_DIM), zz),
            pl.BlockSpec((1, 8 * EDGE_DIM), zz),
        ],
        out_specs=[
            pl.BlockSpec((N8, 8 * NODE_DIM), zz),
            pl.BlockSpec((N8, 8 * EDGE_DIM), zz),
            pl.BlockSpec((N8, 8 * EDGE_DIM), zz),
            pl.BlockSpec((_BE, 8 * EDGE_DIM), eb),
        ],
        out_shape=[
            jax.ShapeDtypeStruct((N8, 8 * NODE_DIM), jnp.float32),
            jax.ShapeDtypeStruct((N8, 8 * EDGE_DIM), jnp.float32),
            jax.ShapeDtypeStruct((N8, 8 * EDGE_DIM), jnp.float32),
            jax.ShapeDtypeStruct((E8, 8 * EDGE_DIM), jnp.float32),
        ],
    )(h8, agg8, wh_k, wa_k, b8.reshape(1, -1), ws_k, wd_k,
      e8, we_k, be8.reshape(1, -1))


def _node_update_body(h8_ref, agg8_ref, wh_ref, wa_ref, b_ref, o_ref):
    a = agg8_ref[0] + agg8_ref[1]
    acc = jnp.dot(h8_ref[...], wh_ref[...], preferred_element_type=jnp.float32)
    acc = acc + jnp.dot(a, wa_ref[...], preferred_element_type=jnp.float32)
    o_ref[...] = jnp.maximum(acc + b_ref[...], 0.0)


def _node_update(h8, agg8, wh_k, wa_k, b8):
    return pl.pallas_call(
        _node_update_body,
        grid=(1,),
        in_specs=[
            pl.BlockSpec((N8, 8 * NODE_DIM), lambda i: (0, 0)),
            pl.BlockSpec((2, N8, 8 * EDGE_DIM), lambda i: (0, 0, 0)),
            pl.BlockSpec((8 * NODE_DIM, 8 * NODE_DIM), lambda i: (0, 0)),
            pl.BlockSpec((8 * EDGE_DIM, 8 * NODE_DIM), lambda i: (0, 0)),
            pl.BlockSpec((1, 8 * NODE_DIM), lambda i: (0, 0)),
        ],
        out_specs=pl.BlockSpec((N8, 8 * NODE_DIM), lambda i: (0, 0)),
        out_shape=jax.ShapeDtypeStruct((N8, 8 * NODE_DIM), jnp.float32),
    )(h8, agg8, wh_k, wa_k, b8.reshape(1, -1))


def _readout_body(gi_ref, h_ref, wg_ref, bg_ref, wo_ref, o_ref, pooled_ref):
    i = pl.program_id(0)
    ng = pl.num_programs(0)

    @pl.when(i == 0)
    def _init():
        pooled_ref[...] = jnp.zeros_like(pooled_ref)

    gi = gi_ref[...]
    iota = lax.broadcasted_iota(jnp.int32, (gi.shape[0], G), 1).astype(jnp.float32)
    onehot = (gi == iota).astype(jnp.float32)
    contrib = lax.dot_general(onehot, h_ref[...],
                              (((0,), (0,)), ((), ())),
                              preferred_element_type=jnp.float32)
    pooled_ref[...] += contrib

    @pl.when(i == ng - 1)
    def _fin():
        g = jnp.maximum(
            jnp.dot(pooled_ref[...], wg_ref[...],
                    preferred_element_type=jnp.float32) + bg_ref[...], 0.0)
        o_ref[...] = jnp.dot(g, wo_ref[...],
                             preferred_element_type=jnp.float32)


def _readout(gi_f, h, wg, bg, wo, block_rows=1000):
    grid = N // block_rows
    return pl.pallas_call(
        _readout_body,
        grid=(grid,),
        in_specs=[
            pl.BlockSpec((block_rows, 1), lambda i: (i, 0)),
            pl.BlockSpec((block_rows, NODE_DIM), lambda i: (i, 0)),
            pl.BlockSpec((NODE_DIM, G), lambda i: (0, 0)),
            pl.BlockSpec((1, G), lambda i: (0, 0)),
            pl.BlockSpec((G, 256), lambda i: (0, 0)),
        ],
        out_specs=pl.BlockSpec((G, 256), lambda i: (0, 0)),
        out_shape=jax.ShapeDtypeStruct((G, 256), jnp.float32),
        scratch_shapes=[pltpu.VMEM((G, G), jnp.float32)],
    )(gi_f, h, wg, bg.reshape(1, -1), wo)


def _make_edge_body(write_eout):
    def body(ei_h, ps_h, pd_h, t_h, *rest):
        if write_eout:
            eout_h, agg_h = rest[0], rest[1]
            scratch = rest[2:]
        else:
            agg_h = rest[0]
            scratch = rest[1:]
        (sidxA, didxA, psvA, pdvA, tvA,
         sidxB, didxB, psvB, pdvB, tvB,
         ev, obuf, agg_sh, semA, semB) = scratch
        c = lax.axis_index("c")
        s = lax.axis_index("s")
        wid = s * _NC + c
        CR = CH // 8

        def fire(j, sidx, didx, psv, pdv, tv, sem):
            cid = j * _NW + wid

            @pl.when(cid < NCHUNK)
            def _():
                pltpu.sync_copy(ei_h.at[0, pl.ds(cid * CH, CH)], sidx)
                pltpu.sync_copy(ei_h.at[1, pl.ds(cid * CH, CH)], didx)
                pltpu.async_copy(ps_h.at[sidx], psv, sem)
                pltpu.async_copy(pd_h.at[didx], pdv, sem)
                pltpu.async_copy(t_h.at[pl.ds(cid * CH, CH), :], tv, sem)

        def drain(j, psv, pdv, tv, sem):
            cid = j * _NW + wid

            @pl.when(cid < NCHUNK)
            def _():
                pltpu.make_async_copy(ps_h.at[pl.ds(0, CH), :],
                                      psv, sem).wait()
                pltpu.make_async_copy(ps_h.at[pl.ds(0, CH), :],
                                      pdv, sem).wait()
                pltpu.make_async_copy(t_h.at[pl.ds(0, CH), :], tv, sem).wait()

        def process(j, didx, psv, pdv, tv):
            cid = j * _NW + wid

            @pl.when(cid < NCHUNK)
            def _():
                def _row(i, carry):
                    tv[i, :] = jnp.maximum(psv[i, :] + pdv[i, :] + tv[i, :],
                                           0.0)
                    return carry
                lax.fori_loop(0, CH, _row, 0)
                if write_eout:
                    pltpu.sync_copy(tv, eout_h.at[pl.ds(cid * CH, CH), :])
                pltpu.sync_copy(tv, agg_sh.at[didx], add=True)

        fire(0, sidxA, didxA, psvA, pdvA, tvA, semA)

        def _zrow(i, carry):
            obuf[i, :] = jnp.zeros((16,), jnp.float32)
            return carry
        lax.fori_loop(0, ROWS_PER_SUB, _zrow, 0)
        pltpu.sync_copy(obuf,
                        agg_sh.at[pl.ds(s * ROWS_PER_SUB, ROWS_PER_SUB), :])
        plsc.subcore_barrier()

        def loop(j2, carry):
            jA = 2 * j2
            jB = jA + 1
            fire(jB, sidxB, didxB, psvB, pdvB, tvB, semB)
            drain(jA, psvA, pdvA, tvA, semA)
            process(jA, didxA, psvA, pdvA, tvA)
            fire(jA + 2, sidxA, didxA, psvA, pdvA, tvA, semA)
            drain(jB, psvB, pdvB, tvB, semB)
            process(jB, didxB, psvB, pdvB, tvB)
            return carry
        lax.fori_loop(0, NJ // 2, loop, 0)

        plsc.subcore_barrier()
        pltpu.sync_copy(agg_sh.at[pl.ds(s * ROWS_PER_SUB, ROWS_PER_SUB), :],
                        obuf)
        pltpu.sync_copy(obuf,
                        agg_h.at[c, pl.ds(s * ROWS_PER_SUB, ROWS_PER_SUB), :])
    return body


def _edge_stage(ei2, ps, pd, t, write_eout=True):
    mesh = plsc.VectorSubcoreMesh(core_axis_name="c", subcore_axis_name="s",
                                  num_cores=_NC, num_subcores=_NS)
    agg_ty = jax.ShapeDtypeStruct((_NC, N, EDGE_DIM), jnp.float32)
    if write_eout:
        out_type = (jax.ShapeDtypeStruct((E, EDGE_DIM), jnp.float32),
                    agg_ty)
    else:
        out_type = (agg_ty,)
    f = pl.kernel(
        _make_edge_body(write_eout),
        out_type=out_type,
        mesh=mesh,
        compiler_params=pltpu.CompilerParams(use_tc_tiling_on_sc=False),
        scratch_types=[
            pltpu.VMEM((CH,), jnp.int32),
            pltpu.VMEM((CH,), jnp.int32),
            pltpu.VMEM((CH, EDGE_DIM), jnp.float32),
            pltpu.VMEM((CH, EDGE_DIM), jnp.float32),
            pltpu.VMEM((CH, EDGE_DIM), jnp.float32),
            pltpu.VMEM((CH,), jnp.int32),
            pltpu.VMEM((CH,), jnp.int32),
            pltpu.VMEM((CH, EDGE_DIM), jnp.float32),
            pltpu.VMEM((CH, EDGE_DIM), jnp.float32),
            pltpu.VMEM((CH, EDGE_DIM), jnp.float32),
            pltpu.VMEM((CH, EDGE_DIM), jnp.float32),
            pltpu.VMEM((ROWS_PER_SUB, EDGE_DIM), jnp.float32),
            pltpu.VMEM_SHARED((N, EDGE_DIM), jnp.float32),
            pltpu.SemaphoreType.DMA,
            pltpu.SemaphoreType.DMA,
        ],
    )
    return f(ei2, ps, pd, t)


def kernel(node_features, edge_features, edge_index, graph_index,
           W_atom, b_atom, W_bond, b_bond, W_edge, b_edge,
           W_node, b_node, W_graph, b_graph, W_out):
    ei2 = edge_index.astype(jnp.int32)
    eye8 = jnp.eye(8, dtype=jnp.float32)

    def ekron(w):
        return jnp.kron(eye8, w)

    ws_k = [ekron(W_edge[l, :NODE_DIM, :]) for l in range(L)]
    wd_k = [ekron(W_edge[l, NODE_DIM:2 * NODE_DIM, :]) for l in range(L)]
    we8 = [ekron(W_edge[l, 2 * NODE_DIM:, :]) for l in range(L)]
    be8 = [jnp.tile(b_edge[l], 8) for l in range(L)]
    wh_k = [ekron(W_node[l, :NODE_DIM, :]) for l in range(L)]
    wa_k = [ekron(W_node[l, NODE_DIM:, :]) for l in range(L)]
    bn8 = [jnp.tile(b_node[l], 8) for l in range(L)]

    nf8 = node_features.reshape(N8, 8 * NODE_DIM)
    wat_k = ekron(W_atom)
    ba8 = jnp.tile(b_atom, 8)
    ws0c = ekron(W_atom @ W_edge[0, :NODE_DIM, :])
    wd0c = ekron(W_atom @ W_edge[0, NODE_DIM:2 * NODE_DIM, :])
    bs0 = jnp.tile(b_atom @ W_edge[0, :NODE_DIM, :], 8)
    bd0 = jnp.tile(b_atom @ W_edge[0, NODE_DIM:2 * NODE_DIM, :], 8)
    ef8 = edge_features.reshape(E8, 8 * EDGE_DIM)
    wb8 = ekron(W_bond)
    bb8 = jnp.tile(b_bond, 8)
    w0c = wb8 @ we8[0]
    b0c = bb8 @ we8[0] + be8[0]
    h8, ps8, pd8, t8 = _embedproj_t0(nf8, wat_k, ba8, ws0c, wd0c, bs0, bd0,
                                     ef8, w0c, b0c)

    for l in range(L):
        last = l == L - 1
        outs = _edge_stage(ei2,
                           ps8.reshape(N, EDGE_DIM),
                           pd8.reshape(N, EDGE_DIM),
                           t8.reshape(E, EDGE_DIM),
                           write_eout=not last)
        if last:
            (aggp,) = outs
            h8 = _node_update(h8, aggp.reshape(_NC, N8, 8 * EDGE_DIM),
                              wh_k[l], wa_k[l], bn8[l])
        else:
            e_new, aggp = outs
            h8, ps8, pd8, t8 = _updproj_tmsg(
                h8, aggp.reshape(_NC, N8, 8 * EDGE_DIM),
                wh_k[l], wa_k[l], bn8[l],
                ws_k[l + 1], wd_k[l + 1],
                e_new.reshape(E8, 8 * EDGE_DIM), we8[l + 1], be8[l + 1])

    gi_f = graph_index.astype(jnp.float32).reshape(N, 1)
    return _readout(gi_f, h8.reshape(N, NODE_DIM), W_graph, b_graph, W_out)


# speedup vs baseline: 1.0685x; 1.0423x over previous
import jax
import jax.numpy as jnp
from jax import lax
from jax.experimental import pallas as pl
from jax.experimental.pallas import tpu as pltpu
from jax.experimental.pallas import tpu_sc as plsc

N = 10000
E = 320000
G = 128
NODE_DIM = 128
EDGE_DIM = 16
L = 3

_NC = 2
_NS = 16
_NW = _NC * _NS
CH = 1000
NCHUNK = E // CH
NJ = (NCHUNK + _NW - 1) // _NW
ROWS_PER_SUB = N // _NS
N8 = N // 8
E8 = E // 8


def _mmb_body(x_ref, w_ref, b_ref, o_ref):
    o_ref[...] = jnp.dot(x_ref[...], w_ref[...],
                         preferred_element_type=jnp.float32) + b_ref[...]


def _matmul_bias(x, w, b, block_rows):
    rows = x.shape[0]
    grid = rows // block_rows
    return pl.pallas_call(
        _mmb_body,
        grid=(grid,),
        in_specs=[
            pl.BlockSpec((block_rows, x.shape[1]), lambda i: (i, 0)),
            pl.BlockSpec(w.shape, lambda i: (0, 0)),
            pl.BlockSpec((1, w.shape[1]), lambda i: (0, 0)),
        ],
        out_specs=pl.BlockSpec((block_rows, w.shape[1]), lambda i: (i, 0)),
        out_shape=jax.ShapeDtypeStruct((rows, w.shape[1]), jnp.float32),
    )(x, w, b.reshape(1, -1))


_BE = 2000
_NEB = E8 // _BE


def _embedproj_t0_body(nf8_ref, wat_ref, ba_ref, ws_ref, wd_ref, bs_ref,
                       bd_ref, ef_ref, w0_ref, b0_ref,
                       h8_ref, ps_ref, pd_ref, t_ref):
    i = pl.program_id(0)

    @pl.when(i == 0)
    def _node():
        nf8 = nf8_ref[...]
        h8_ref[...] = jnp.dot(nf8, wat_ref[...],
                              preferred_element_type=jnp.float32) + ba_ref[...]
        ps_ref[...] = jnp.dot(nf8, ws_ref[...],
                              preferred_element_type=jnp.float32) + bs_ref[...]
        pd_ref[...] = jnp.dot(nf8, wd_ref[...],
                              preferred_element_type=jnp.float32) + bd_ref[...]

    @pl.when(i > 0)
    def _edge():
        t_ref[...] = jnp.dot(ef_ref[...], w0_ref[...],
                             preferred_element_type=jnp.float32) + b0_ref[...]


def _embedproj_t0(nf8, wat_k, ba8, ws_c, wd_c, bs0, bd0, ef8, w0c, b0c):
    zz = lambda i: (0, 0)
    eb = lambda i: (jnp.maximum(i - 1, 0), 0)
    return pl.pallas_call(
        _embedproj_t0_body,
        grid=(1 + _NEB,),
        in_specs=[
            pl.BlockSpec((N8, 8 * NODE_DIM), zz),
            pl.BlockSpec((8 * NODE_DIM, 8 * NODE_DIM), zz),
            pl.BlockSpec((1, 8 * NODE_DIM), zz),
            pl.BlockSpec((8 * NODE_DIM, 8 * EDGE_DIM), zz),
            pl.BlockSpec((8 * NODE_DIM, 8 * EDGE_DIM), zz),
            pl.BlockSpec((1, 8 * EDGE_DIM), zz),
            pl.BlockSpec((1, 8 * EDGE_DIM), zz),
            pl.BlockSpec((_BE, 8 * EDGE_DIM), eb),
            pl.BlockSpec((8 * EDGE_DIM, 8 * EDGE_DIM), zz),
            pl.BlockSpec((1, 8 * EDGE_DIM), zz),
        ],
        out_specs=[
            pl.BlockSpec((N8, 8 * NODE_DIM), zz),
            pl.BlockSpec((N8, 8 * EDGE_DIM), zz),
            pl.BlockSpec((N8, 8 * EDGE_DIM), zz),
            pl.BlockSpec((_BE, 8 * EDGE_DIM), eb),
        ],
        out_shape=[
            jax.ShapeDtypeStruct((N8, 8 * NODE_DIM), jnp.float32),
            jax.ShapeDtypeStruct((N8, 8 * EDGE_DIM), jnp.float32),
            jax.ShapeDtypeStruct((N8, 8 * EDGE_DIM), jnp.float32),
            jax.ShapeDtypeStruct((E8, 8 * EDGE_DIM), jnp.float32),
        ],
    )(nf8, wat_k, ba8.reshape(1, -1), ws_c, wd_c,
      bs0.reshape(1, -1), bd0.reshape(1, -1), ef8, w0c, b0c.reshape(1, -1))


def _updproj_tmsg_body(h8_ref, agg8_ref, wh_ref, wa_ref, b_ref, ws_ref,
                       wd_ref, e8_ref, we_ref, be_ref,
                       o_ref, ps_ref, pd_ref, t_ref):
    i = pl.program_id(0)

    @pl.when(i == 0)
    def _node():
        a = agg8_ref[0] + agg8_ref[1]
        acc = jnp.dot(h8_ref[...], wh_ref[...],
                      preferred_element_type=jnp.float32)
        acc = acc + jnp.dot(a, wa_ref[...], preferred_element_type=jnp.float32)
        h8 = jnp.maximum(acc + b_ref[...], 0.0)
        o_ref[...] = h8
        ps_ref[...] = jnp.dot(h8, ws_ref[...],
                              preferred_element_type=jnp.float32)
        pd_ref[...] = jnp.dot(h8, wd_ref[...],
                              preferred_element_type=jnp.float32)

    @pl.when(i > 0)
    def _edge():
        t_ref[...] = jnp.dot(e8_ref[...], we_ref[...],
                             preferred_element_type=jnp.float32) + be_ref[...]


def _updproj_tmsg(h8, agg8, wh_k, wa_k, b8, ws_k, wd_k, e8, we_k, be8):
    zz = lambda i: (0, 0)
    eb = lambda i: (jnp.maximum(i - 1, 0), 0)
    return pl.pallas_call(
        _updproj_tmsg_body,
        grid=(1 + _NEB,),
        in_specs=[
            pl.BlockSpec((N8, 8 * NODE_DIM), zz),
            pl.BlockSpec((2, N8, 8 * EDGE_DIM), lambda i: (0, 0, 0)),
            pl.BlockSpec((8 * NODE_DIM, 8 * NODE_DIM), zz),
            pl.BlockSpec((8 * EDGE_DIM, 8 * NODE_DIM), zz),
            pl.BlockSpec((1, 8 * NODE_DIM), zz),
            pl.BlockSpec((8 * NODE_DIM, 8 * EDGE_DIM), zz),
            pl.BlockSpec((8 * NODE_DIM, 8 * EDGE_DIM), zz),
            pl.BlockSpec((_BE, 8 * EDGE_DIM), eb),
            pl.BlockSpec((8 * EDGE_DIM, 8 * EDGE_DIM), zz),
            pl.BlockSpec((1, 8 * EDGE_DIM), zz),
        ],
        out_specs=[
            pl.BlockSpec((N8, 8 * NODE_DIM), zz),
            pl.BlockSpec((N8, 8 * EDGE_DIM), zz),
            pl.BlockSpec((N8, 8 * EDGE_DIM), zz),
            pl.BlockSpec((_BE, 8 * EDGE_DIM), eb),
        ],
        out_shape=[
            jax.ShapeDtypeStruct((N8, 8 * NODE_DIM), jnp.float32),
            jax.ShapeDtypeStruct((N8, 8 * EDGE_DIM), jnp.float32),
            jax.ShapeDtypeStruct((N8, 8 * EDGE_DIM), jnp.float32),
            jax.ShapeDtypeStruct((E8, 8 * EDGE_DIM), jnp.float32),
        ],
    )(h8, agg8, wh_k, wa_k, b8.reshape(1, -1), ws_k, wd_k,
      e8, we_k, be8.reshape(1, -1))


def _node_update_body(h8_ref, agg8_ref, wh_ref, wa_ref, b_ref, o_ref):
    a = agg8_ref[0] + agg8_ref[1]
    acc = jnp.dot(h8_ref[...], wh_ref[...], preferred_element_type=jnp.float32)
    acc = acc + jnp.dot(a, wa_ref[...], preferred_element_type=jnp.float32)
    o_ref[...] = jnp.maximum(acc + b_ref[...], 0.0)


def _node_update(h8, agg8, wh_k, wa_k, b8):
    return pl.pallas_call(
        _node_update_body,
        grid=(1,),
        in_specs=[
            pl.BlockSpec((N8, 8 * NODE_DIM), lambda i: (0, 0)),
            pl.BlockSpec((2, N8, 8 * EDGE_DIM), lambda i: (0, 0, 0)),
            pl.BlockSpec((8 * NODE_DIM, 8 * NODE_DIM), lambda i: (0, 0)),
            pl.BlockSpec((8 * EDGE_DIM, 8 * NODE_DIM), lambda i: (0, 0)),
            pl.BlockSpec((1, 8 * NODE_DIM), lambda i: (0, 0)),
        ],
        out_specs=pl.BlockSpec((N8, 8 * NODE_DIM), lambda i: (0, 0)),
        out_shape=jax.ShapeDtypeStruct((N8, 8 * NODE_DIM), jnp.float32),
    )(h8, agg8, wh_k, wa_k, b8.reshape(1, -1))


def _readout_body(gi_ref, h_ref, wg_ref, bg_ref, wo_ref, o_ref, pooled_ref):
    i = pl.program_id(0)
    ng = pl.num_programs(0)

    @pl.when(i == 0)
    def _init():
        pooled_ref[...] = jnp.zeros_like(pooled_ref)

    gi = gi_ref[...]
    iota = lax.broadcasted_iota(jnp.int32, (gi.shape[0], G), 1).astype(jnp.float32)
    onehot = (gi == iota).astype(jnp.float32)
    contrib = lax.dot_general(onehot, h_ref[...],
                              (((0,), (0,)), ((), ())),
                              preferred_element_type=jnp.float32)
    pooled_ref[...] += contrib

    @pl.when(i == ng - 1)
    def _fin():
        g = jnp.maximum(
            jnp.dot(pooled_ref[...], wg_ref[...],
                    preferred_element_type=jnp.float32) + bg_ref[...], 0.0)
        o_ref[...] = jnp.dot(g, wo_ref[...],
                             preferred_element_type=jnp.float32)


def _readout(gi_f, h, wg, bg, wo, block_rows=1000):
    grid = N // block_rows
    return pl.pallas_call(
        _readout_body,
        grid=(grid,),
        in_specs=[
            pl.BlockSpec((block_rows, 1), lambda i: (i, 0)),
            pl.BlockSpec((block_rows, NODE_DIM), lambda i: (i, 0)),
            pl.BlockSpec((NODE_DIM, G), lambda i: (0, 0)),
            pl.BlockSpec((1, G), lambda i: (0, 0)),
            pl.BlockSpec((G, 256), lambda i: (0, 0)),
        ],
        out_specs=pl.BlockSpec((G, 256), lambda i: (0, 0)),
        out_shape=jax.ShapeDtypeStruct((G, 256), jnp.float32),
        scratch_shapes=[pltpu.VMEM((G, G), jnp.float32)],
    )(gi_f, h, wg, bg.reshape(1, -1), wo)


def _make_edge_body(write_eout):
    def body(ei_h, ps_h, pd_h, t_h, *rest):
        if write_eout:
            eout_h, agg_h = rest[0], rest[1]
            scratch = rest[2:]
        else:
            agg_h = rest[0]
            scratch = rest[1:]
        (sidxA, didxA, psvA, pdvA, tvA,
         sidxB, didxB, psvB, pdvB, tvB,
         ev, obuf, agg_sh, semA, semB) = scratch
        c = lax.axis_index("c")
        s = lax.axis_index("s")
        wid = s * _NC + c
        CR = CH // 8

        def fire(j, sidx, didx, psv, pdv, tv, sem):
            cid = j * _NW + wid

            @pl.when(cid < NCHUNK)
            def _():
                pltpu.sync_copy(ei_h.at[0, pl.ds(cid * CH, CH)], sidx)
                pltpu.sync_copy(ei_h.at[1, pl.ds(cid * CH, CH)], didx)
                pltpu.async_copy(ps_h.at[sidx], psv, sem)
                pltpu.async_copy(pd_h.at[didx], pdv, sem)
                pltpu.async_copy(t_h.at[pl.ds(cid * CH, CH), :], tv, sem)

        def drain(j, psv, pdv, tv, sem):
            cid = j * _NW + wid

            @pl.when(cid < NCHUNK)
            def _():
                pltpu.make_async_copy(ps_h.at[pl.ds(0, CH), :],
                                      psv, sem).wait()
                pltpu.make_async_copy(ps_h.at[pl.ds(0, CH), :],
                                      pdv, sem).wait()
                pltpu.make_async_copy(t_h.at[pl.ds(0, CH), :], tv, sem).wait()

        def process(j, didx, psv, pdv, tv):
            cid = j * _NW + wid

            @pl.when(cid < NCHUNK)
            def _():
                def _row(i, carry):
                    tv[i, :] = jnp.maximum(psv[i, :] + pdv[i, :] + tv[i, :],
                                           0.0)
                    return carry
                lax.fori_loop(0, CH, _row, 0)
                if write_eout:
                    pltpu.sync_copy(tv, eout_h.at[pl.ds(cid * CH, CH), :])
                pltpu.sync_copy(tv, agg_sh.at[didx], add=True)

        fire(0, sidxA, didxA, psvA, pdvA, tvA, semA)

        def _zrow(i, carry):
            obuf[i, :] = jnp.zeros((16,), jnp.float32)
            return carry
        lax.fori_loop(0, ROWS_PER_SUB, _zrow, 0)
        pltpu.sync_copy(obuf,
                        agg_sh.at[pl.ds(s * ROWS_PER_SUB, ROWS_PER_SUB), :])
        plsc.subcore_barrier()

        def loop(j2, carry):
            jA = 2 * j2
            jB = jA + 1
            fire(jB, sidxB, didxB, psvB, pdvB, tvB, semB)
            drain(jA, psvA, pdvA, tvA, semA)
            process(jA, didxA, psvA, pdvA, tvA)
            fire(jA + 2, sidxA, didxA, psvA, pdvA, tvA, semA)
            drain(jB, psvB, pdvB, tvB, semB)
            process(jB, didxB, psvB, pdvB, tvB)
            return carry
        lax.fori_loop(0, NJ // 2, loop, 0)

        plsc.subcore_barrier()
        pltpu.sync_copy(agg_sh.at[pl.ds(s * ROWS_PER_SUB, ROWS_PER_SUB), :],
                        obuf)
        pltpu.sync_copy(obuf,
                        agg_h.at[c, pl.ds(s * ROWS_PER_SUB, ROWS_PER_SUB), :])
    return body


def _edge_stage(ei2, ps, pd, t, write_eout=True):
    mesh = plsc.VectorSubcoreMesh(core_axis_name="c", subcore_axis_name="s",
                                  num_cores=_NC, num_subcores=_NS)
    agg_ty = jax.ShapeDtypeStruct((_NC, N, EDGE_DIM), jnp.float32)
    if write_eout:
        out_type = (jax.ShapeDtypeStruct((E, EDGE_DIM), jnp.float32),
                    agg_ty)
    else:
        out_type = (agg_ty,)
    f = pl.kernel(
        _make_edge_body(write_eout),
        out_type=out_type,
        mesh=mesh,
        compiler_params=pltpu.CompilerParams(use_tc_tiling_on_sc=False),
        scratch_types=[
            pltpu.VMEM((CH,), jnp.int32),
            pltpu.VMEM((CH,), jnp.int32),
            pltpu.VMEM((CH, EDGE_DIM), jnp.float32),
            pltpu.VMEM((CH, EDGE_DIM), jnp.float32),
            pltpu.VMEM((CH, EDGE_DIM), jnp.float32),
            pltpu.VMEM((CH,), jnp.int32),
            pltpu.VMEM((CH,), jnp.int32),
            pltpu.VMEM((CH, EDGE_DIM), jnp.float32),
            pltpu.VMEM((CH, EDGE_DIM), jnp.float32),
            pltpu.VMEM((CH, EDGE_DIM), jnp.float32),
            pltpu.VMEM((CH, EDGE_DIM), jnp.float32),
            pltpu.VMEM((ROWS_PER_SUB, EDGE_DIM), jnp.float32),
            pltpu.VMEM_SHARED((N, EDGE_DIM), jnp.float32),
            pltpu.SemaphoreType.DMA,
            pltpu.SemaphoreType.DMA,
        ],
    )
    return f(ei2, ps, pd, t)


def kernel(node_features, edge_features, edge_index, graph_index,
           W_atom, b_atom, W_bond, b_bond, W_edge, b_edge,
           W_node, b_node, W_graph, b_graph, W_out):
    ei2 = edge_index.astype(jnp.int32)
    eye8 = jnp.eye(8, dtype=jnp.float32)

    def ekron(w):
        return jnp.kron(eye8, w)

    ws_k = [ekron(W_edge[l, :NODE_DIM, :]) for l in range(L)]
    wd_k = [ekron(W_edge[l, NODE_DIM:2 * NODE_DIM, :]) for l in range(L)]
    we8 = [ekron(W_edge[l, 2 * NODE_DIM:, :]) for l in range(L)]
    be8 = [jnp.tile(b_edge[l], 8) for l in range(L)]
    wh_k = [ekron(W_node[l, :NODE_DIM, :]) for l in range(L)]
    wa_k = [ekron(W_node[l, NODE_DIM:, :]) for l in range(L)]
    bn8 = [jnp.tile(b_node[l], 8) for l in range(L)]

    nf8 = node_features.reshape(N8, 8 * NODE_DIM)
    wat_k = ekron(W_atom)
    ba8 = jnp.tile(b_atom, 8)
    ws0c = ekron(W_atom @ W_edge[0, :NODE_DIM, :])
    wd0c = ekron(W_atom @ W_edge[0, NODE_DIM:2 * NODE_DIM, :])
    bs0 = jnp.tile(b_atom @ W_edge[0, :NODE_DIM, :], 8)
    bd0 = jnp.tile(b_atom @ W_edge[0, NODE_DIM:2 * NODE_DIM, :], 8)
    ef8 = edge_features.reshape(E8, 8 * EDGE_DIM)
    wb8 = ekron(W_bond)
    bb8 = jnp.tile(b_bond, 8)
    w0c = wb8 @ we8[0]
    b0c = bb8 @ we8[0] + be8[0]
    h8, ps8, pd8, t8 = _embedproj_t0(nf8, wat_k, ba8, ws0c, wd0c, bs0, bd0,
                                     ef8, w0c, b0c)

    for l in range(L):
        last = l == L - 1
        outs = _edge_stage(ei2,
                           ps8.reshape(N, EDGE_DIM),
                           pd8.reshape(N, EDGE_DIM),
                           t8.reshape(E, EDGE_DIM),
                           write_eout=not last)
        if last:
            (aggp,) = outs
            h8 = _node_update(h8, aggp.reshape(_NC, N8, 8 * EDGE_DIM),
                              wh_k[l], wa_k[l], bn8[l])
        else:
            e_new, aggp = outs
            h8, ps8, pd8, t8 = _updproj_tmsg(
                h8, aggp.reshape(_NC, N8, 8 * EDGE_DIM),
                wh_k[l], wa_k[l], bn8[l],
                ws_k[l + 1], wd_k[l + 1],
                e_new.reshape(E8, 8 * EDGE_DIM), we8[l + 1], be8[l + 1])

    gi_f = graph_index.astype(jnp.float32).reshape(N, 1)
    return _readout(gi_f, h8.reshape(N, NODE_DIM), W_graph, b_graph, W_out)


# speedup vs baseline: 1.0809x; 1.0116x over previous
import jax
import jax.numpy as jnp
from jax import lax
from jax.experimental import pallas as pl
from jax.experimental.pallas import tpu as pltpu
from jax.experimental.pallas import tpu_sc as plsc

N = 10000
E = 320000
G = 128
NODE_DIM = 128
EDGE_DIM = 16
L = 3

_NC = 2
_NS = 16
_NW = _NC * _NS
CH = 1000
NCHUNK = E // CH
NJ = (NCHUNK + _NW - 1) // _NW
ROWS_PER_SUB = N // _NS
N8 = N // 8
E8 = E // 8


def _mmb_body(x_ref, w_ref, b_ref, o_ref):
    o_ref[...] = jnp.dot(x_ref[...], w_ref[...],
                         preferred_element_type=jnp.float32) + b_ref[...]


def _matmul_bias(x, w, b, block_rows):
    rows = x.shape[0]
    grid = rows // block_rows
    return pl.pallas_call(
        _mmb_body,
        grid=(grid,),
        in_specs=[
            pl.BlockSpec((block_rows, x.shape[1]), lambda i: (i, 0)),
            pl.BlockSpec(w.shape, lambda i: (0, 0)),
            pl.BlockSpec((1, w.shape[1]), lambda i: (0, 0)),
        ],
        out_specs=pl.BlockSpec((block_rows, w.shape[1]), lambda i: (i, 0)),
        out_shape=jax.ShapeDtypeStruct((rows, w.shape[1]), jnp.float32),
    )(x, w, b.reshape(1, -1))


_BE = 2000
_NEB = E8 // _BE


def _embedproj_t0_body(nf8_ref, wat_ref, ba_ref, ws_ref, wd_ref, bs_ref,
                       bd_ref, ef_ref, w0_ref, b0_ref,
                       h8_ref, ps_ref, pd_ref, t_ref):
    i = pl.program_id(0)

    @pl.when(i == 0)
    def _node():
        nf8 = nf8_ref[...]
        h8_ref[...] = jnp.dot(nf8, wat_ref[...],
                              preferred_element_type=jnp.float32) + ba_ref[...]
        ps_ref[...] = jnp.dot(nf8, ws_ref[...],
                              preferred_element_type=jnp.float32) + bs_ref[...]
        pd_ref[...] = jnp.dot(nf8, wd_ref[...],
                              preferred_element_type=jnp.float32) + bd_ref[...]

    @pl.when(i > 0)
    def _edge():
        t_ref[...] = jnp.dot(ef_ref[...], w0_ref[...],
                             preferred_element_type=jnp.float32) + b0_ref[...]


def _embedproj_t0(nf8, wat_k, ba8, ws_c, wd_c, bs0, bd0, ef8, w0c, b0c):
    zz = lambda i: (0, 0)
    eb = lambda i: (jnp.maximum(i - 1, 0), 0)
    return pl.pallas_call(
        _embedproj_t0_body,
        grid=(1 + _NEB,),
        in_specs=[
            pl.BlockSpec((N8, 8 * NODE_DIM), zz),
            pl.BlockSpec((8 * NODE_DIM, 8 * NODE_DIM), zz),
            pl.BlockSpec((1, 8 * NODE_DIM), zz),
            pl.BlockSpec((8 * NODE_DIM, 8 * EDGE_DIM), zz),
            pl.BlockSpec((8 * NODE_DIM, 8 * EDGE_DIM), zz),
            pl.BlockSpec((1, 8 * EDGE_DIM), zz),
            pl.BlockSpec((1, 8 * EDGE_DIM), zz),
            pl.BlockSpec((_BE, 8 * EDGE_DIM), eb),
            pl.BlockSpec((8 * EDGE_DIM, 8 * EDGE_DIM), zz),
            pl.BlockSpec((1, 8 * EDGE_DIM), zz),
        ],
        out_specs=[
            pl.BlockSpec((N8, 8 * NODE_DIM), zz),
            pl.BlockSpec((N8, 8 * EDGE_DIM), zz),
            pl.BlockSpec((N8, 8 * EDGE_DIM), zz),
            pl.BlockSpec((_BE, 8 * EDGE_DIM), eb),
        ],
        out_shape=[
            jax.ShapeDtypeStruct((N8, 8 * NODE_DIM), jnp.float32),
            jax.ShapeDtypeStruct((N8, 8 * EDGE_DIM), jnp.float32),
            jax.ShapeDtypeStruct((N8, 8 * EDGE_DIM), jnp.float32),
            jax.ShapeDtypeStruct((E8, 8 * EDGE_DIM), jnp.float32),
        ],
    )(nf8, wat_k, ba8.reshape(1, -1), ws_c, wd_c,
      bs0.reshape(1, -1), bd0.reshape(1, -1), ef8, w0c, b0c.reshape(1, -1))


def _updproj_tmsg_body(h8_ref, agg8_ref, wh_ref, wa_ref, b_ref, ws_ref,
                       wd_ref, e8_ref, we_ref, be_ref,
                       o_ref, ps_ref, pd_ref, t_ref):
    i = pl.program_id(0)

    @pl.when(i == 0)
    def _node():
        a = agg8_ref[0] + agg8_ref[1]
        acc = jnp.dot(h8_ref[...], wh_ref[...],
                      preferred_element_type=jnp.float32)
        acc = acc + jnp.dot(a, wa_ref[...], preferred_element_type=jnp.float32)
        h8 = jnp.maximum(acc + b_ref[...], 0.0)
        o_ref[...] = h8
        ps_ref[...] = jnp.dot(h8, ws_ref[...],
                              preferred_element_type=jnp.float32)
        pd_ref[...] = jnp.dot(h8, wd_ref[...],
                              preferred_element_type=jnp.float32)

    @pl.when(i > 0)
    def _edge():
        t_ref[...] = jnp.dot(e8_ref[...], we_ref[...],
                             preferred_element_type=jnp.float32) + be_ref[...]


def _updproj_tmsg(h8, agg8, wh_k, wa_k, b8, ws_k, wd_k, e8, we_k, be8):
    zz = lambda i: (0, 0)
    eb = lambda i: (jnp.maximum(i - 1, 0), 0)
    return pl.pallas_call(
        _updproj_tmsg_body,
        grid=(1 + _NEB,),
        in_specs=[
            pl.BlockSpec((N8, 8 * NODE_DIM), zz),
            pl.BlockSpec((2, N8, 8 * EDGE_DIM), lambda i: (0, 0, 0)),
            pl.BlockSpec((8 * NODE_DIM, 8 * NODE_DIM), zz),
            pl.BlockSpec((8 * EDGE_DIM, 8 * NODE_DIM), zz),
            pl.BlockSpec((1, 8 * NODE_DIM), zz),
            pl.BlockSpec((8 * NODE_DIM, 8 * EDGE_DIM), zz),
            pl.BlockSpec((8 * NODE_DIM, 8 * EDGE_DIM), zz),
            pl.BlockSpec((_BE, 8 * EDGE_DIM), eb),
            pl.BlockSpec((8 * EDGE_DIM, 8 * EDGE_DIM), zz),
            pl.BlockSpec((1, 8 * EDGE_DIM), zz),
        ],
        out_specs=[
            pl.BlockSpec((N8, 8 * NODE_DIM), zz),
            pl.BlockSpec((N8, 8 * EDGE_DIM), zz),
            pl.BlockSpec((N8, 8 * EDGE_DIM), zz),
            pl.BlockSpec((_BE, 8 * EDGE_DIM), eb),
        ],
        out_shape=[
            jax.ShapeDtypeStruct((N8, 8 * NODE_DIM), jnp.float32),
            jax.ShapeDtypeStruct((N8, 8 * EDGE_DIM), jnp.float32),
            jax.ShapeDtypeStruct((N8, 8 * EDGE_DIM), jnp.float32),
            jax.ShapeDtypeStruct((E8, 8 * EDGE_DIM), jnp.float32),
        ],
    )(h8, agg8, wh_k, wa_k, b8.reshape(1, -1), ws_k, wd_k,
      e8, we_k, be8.reshape(1, -1))


def _node_update_body(h8_ref, agg8_ref, wh_ref, wa_ref, b_ref, o_ref):
    a = agg8_ref[0] + agg8_ref[1]
    acc = jnp.dot(h8_ref[...], wh_ref[...], preferred_element_type=jnp.float32)
    acc = acc + jnp.dot(a, wa_ref[...], preferred_element_type=jnp.float32)
    o_ref[...] = jnp.maximum(acc + b_ref[...], 0.0)


def _node_update(h8, agg8, wh_k, wa_k, b8):
    return pl.pallas_call(
        _node_update_body,
        grid=(1,),
        in_specs=[
            pl.BlockSpec((N8, 8 * NODE_DIM), lambda i: (0, 0)),
            pl.BlockSpec((2, N8, 8 * EDGE_DIM), lambda i: (0, 0, 0)),
            pl.BlockSpec((8 * NODE_DIM, 8 * NODE_DIM), lambda i: (0, 0)),
            pl.BlockSpec((8 * EDGE_DIM, 8 * NODE_DIM), lambda i: (0, 0)),
            pl.BlockSpec((1, 8 * NODE_DIM), lambda i: (0, 0)),
        ],
        out_specs=pl.BlockSpec((N8, 8 * NODE_DIM), lambda i: (0, 0)),
        out_shape=jax.ShapeDtypeStruct((N8, 8 * NODE_DIM), jnp.float32),
    )(h8, agg8, wh_k, wa_k, b8.reshape(1, -1))


def _readout_body(gi_ref, h_ref, wg_ref, bg_ref, wo_ref, o_ref, pooled_ref):
    i = pl.program_id(0)
    ng = pl.num_programs(0)

    @pl.when(i == 0)
    def _init():
        pooled_ref[...] = jnp.zeros_like(pooled_ref)

    gi = gi_ref[...]
    iota = lax.broadcasted_iota(jnp.int32, (gi.shape[0], G), 1).astype(jnp.float32)
    onehot = (gi == iota).astype(jnp.float32)
    contrib = lax.dot_general(onehot, h_ref[...],
                              (((0,), (0,)), ((), ())),
                              preferred_element_type=jnp.float32)
    pooled_ref[...] += contrib

    @pl.when(i == ng - 1)
    def _fin():
        g = jnp.maximum(
            jnp.dot(pooled_ref[...], wg_ref[...],
                    preferred_element_type=jnp.float32) + bg_ref[...], 0.0)
        o_ref[...] = jnp.dot(g, wo_ref[...],
                             preferred_element_type=jnp.float32)


def _readout(gi_f, h, wg, bg, wo, block_rows=1000):
    grid = N // block_rows
    return pl.pallas_call(
        _readout_body,
        grid=(grid,),
        in_specs=[
            pl.BlockSpec((block_rows, 1), lambda i: (i, 0)),
            pl.BlockSpec((block_rows, NODE_DIM), lambda i: (i, 0)),
            pl.BlockSpec((NODE_DIM, G), lambda i: (0, 0)),
            pl.BlockSpec((1, G), lambda i: (0, 0)),
            pl.BlockSpec((G, 256), lambda i: (0, 0)),
        ],
        out_specs=pl.BlockSpec((G, 256), lambda i: (0, 0)),
        out_shape=jax.ShapeDtypeStruct((G, 256), jnp.float32),
        scratch_shapes=[pltpu.VMEM((G, G), jnp.float32)],
    )(gi_f, h, wg, bg.reshape(1, -1), wo)


def _make_edge_body(write_eout):
    def body(ei_h, ps_h, pd_h, t_h, *rest):
        if write_eout:
            eout_h, agg_h = rest[0], rest[1]
            scratch = rest[2:]
        else:
            agg_h = rest[0]
            scratch = rest[1:]
        (sidxA, didxA, psvA, pdvA, tvA,
         sidxB, didxB, psvB, pdvB, tvB,
         ev, obuf, agg_sh, semA, semB, wsemA, wsemB) = scratch
        c = lax.axis_index("c")
        s = lax.axis_index("s")
        wid = s * _NC + c
        CR = CH // 8

        def drain_w(j, tv, wsem):
            if not write_eout:
                return
            cid = j * _NW + wid

            @pl.when((j >= 0) & (cid < NCHUNK))
            def _():
                pltpu.make_async_copy(tv, eout_h.at[pl.ds(0, CH), :],
                                      wsem).wait()

        def fire(j, sidx, didx, psv, pdv, tv, sem, wsem):
            cid = j * _NW + wid
            drain_w(j - 2, tv, wsem)

            @pl.when(cid < NCHUNK)
            def _():
                pltpu.sync_copy(ei_h.at[0, pl.ds(cid * CH, CH)], sidx)
                pltpu.sync_copy(ei_h.at[1, pl.ds(cid * CH, CH)], didx)
                pltpu.async_copy(ps_h.at[sidx], psv, sem)
                pltpu.async_copy(pd_h.at[didx], pdv, sem)
                pltpu.async_copy(t_h.at[pl.ds(cid * CH, CH), :], tv, sem)

        def drain(j, psv, pdv, tv, sem):
            cid = j * _NW + wid

            @pl.when(cid < NCHUNK)
            def _():
                pltpu.make_async_copy(ps_h.at[pl.ds(0, CH), :],
                                      psv, sem).wait()
                pltpu.make_async_copy(ps_h.at[pl.ds(0, CH), :],
                                      pdv, sem).wait()
                pltpu.make_async_copy(t_h.at[pl.ds(0, CH), :], tv, sem).wait()

        def process(j, didx, psv, pdv, tv, wsem):
            cid = j * _NW + wid

            @pl.when(cid < NCHUNK)
            def _():
                def _row(i, carry):
                    tv[i, :] = jnp.maximum(psv[i, :] + pdv[i, :] + tv[i, :],
                                           0.0)
                    return carry
                lax.fori_loop(0, CH, _row, 0)
                if write_eout:
                    pltpu.async_copy(tv, eout_h.at[pl.ds(cid * CH, CH), :],
                                     wsem)
                pltpu.sync_copy(tv, agg_sh.at[didx], add=True)

        fire(0, sidxA, didxA, psvA, pdvA, tvA, semA, wsemA)

        def _zrow(i, carry):
            obuf[i, :] = jnp.zeros((16,), jnp.float32)
            return carry
        lax.fori_loop(0, ROWS_PER_SUB, _zrow, 0)
        pltpu.sync_copy(obuf,
                        agg_sh.at[pl.ds(s * ROWS_PER_SUB, ROWS_PER_SUB), :])
        plsc.subcore_barrier()

        def loop(j2, carry):
            jA = 2 * j2
            jB = jA + 1
            fire(jB, sidxB, didxB, psvB, pdvB, tvB, semB, wsemB)
            drain(jA, psvA, pdvA, tvA, semA)
            process(jA, didxA, psvA, pdvA, tvA, wsemA)
            fire(jA + 2, sidxA, didxA, psvA, pdvA, tvA, semA, wsemA)
            drain(jB, psvB, pdvB, tvB, semB)
            process(jB, didxB, psvB, pdvB, tvB, wsemB)
            return carry
        lax.fori_loop(0, NJ // 2, loop, 0)

        drain_w(NJ - 1, tvB, wsemB)

        plsc.subcore_barrier()
        pltpu.sync_copy(agg_sh.at[pl.ds(s * ROWS_PER_SUB, ROWS_PER_SUB), :],
                        obuf)
        pltpu.sync_copy(obuf,
                        agg_h.at[c, pl.ds(s * ROWS_PER_SUB, ROWS_PER_SUB), :])
    return body


def _edge_stage(ei2, ps, pd, t, write_eout=True):
    mesh = plsc.VectorSubcoreMesh(core_axis_name="c", subcore_axis_name="s",
                                  num_cores=_NC, num_subcores=_NS)
    agg_ty = jax.ShapeDtypeStruct((_NC, N, EDGE_DIM), jnp.float32)
    if write_eout:
        out_type = (jax.ShapeDtypeStruct((E, EDGE_DIM), jnp.float32),
                    agg_ty)
    else:
        out_type = (agg_ty,)
    f = pl.kernel(
        _make_edge_body(write_eout),
        out_type=out_type,
        mesh=mesh,
        compiler_params=pltpu.CompilerParams(use_tc_tiling_on_sc=False),
        scratch_types=[
            pltpu.VMEM((CH,), jnp.int32),
            pltpu.VMEM((CH,), jnp.int32),
            pltpu.VMEM((CH, EDGE_DIM), jnp.float32),
            pltpu.VMEM((CH, EDGE_DIM), jnp.float32),
            pltpu.VMEM((CH, EDGE_DIM), jnp.float32),
            pltpu.VMEM((CH,), jnp.int32),
            pltpu.VMEM((CH,), jnp.int32),
            pltpu.VMEM((CH, EDGE_DIM), jnp.float32),
            pltpu.VMEM((CH, EDGE_DIM), jnp.float32),
            pltpu.VMEM((CH, EDGE_DIM), jnp.float32),
            pltpu.VMEM((CH, EDGE_DIM), jnp.float32),
            pltpu.VMEM((ROWS_PER_SUB, EDGE_DIM), jnp.float32),
            pltpu.VMEM_SHARED((N, EDGE_DIM), jnp.float32),
            pltpu.SemaphoreType.DMA,
            pltpu.SemaphoreType.DMA,
            pltpu.SemaphoreType.DMA,
            pltpu.SemaphoreType.DMA,
        ],
    )
    return f(ei2, ps, pd, t)


def kernel(node_features, edge_features, edge_index, graph_index,
           W_atom, b_atom, W_bond, b_bond, W_edge, b_edge,
           W_node, b_node, W_graph, b_graph, W_out):
    ei2 = edge_index.astype(jnp.int32)
    eye8 = jnp.eye(8, dtype=jnp.float32)

    def ekron(w):
        return jnp.kron(eye8, w)

    ws_k = [ekron(W_edge[l, :NODE_DIM, :]) for l in range(L)]
    wd_k = [ekron(W_edge[l, NODE_DIM:2 * NODE_DIM, :]) for l in range(L)]
    we8 = [ekron(W_edge[l, 2 * NODE_DIM:, :]) for l in range(L)]
    be8 = [jnp.tile(b_edge[l], 8) for l in range(L)]
    wh_k = [ekron(W_node[l, :NODE_DIM, :]) for l in range(L)]
    wa_k = [ekron(W_node[l, NODE_DIM:, :]) for l in range(L)]
    bn8 = [jnp.tile(b_node[l], 8) for l in range(L)]

    nf8 = node_features.reshape(N8, 8 * NODE_DIM)
    wat_k = ekron(W_atom)
    ba8 = jnp.tile(b_atom, 8)
    ws0c = ekron(W_atom @ W_edge[0, :NODE_DIM, :])
    wd0c = ekron(W_atom @ W_edge[0, NODE_DIM:2 * NODE_DIM, :])
    bs0 = jnp.tile(b_atom @ W_edge[0, :NODE_DIM, :], 8)
    bd0 = jnp.tile(b_atom @ W_edge[0, NODE_DIM:2 * NODE_DIM, :], 8)
    ef8 = edge_features.reshape(E8, 8 * EDGE_DIM)
    wb8 = ekron(W_bond)
    bb8 = jnp.tile(b_bond, 8)
    w0c = wb8 @ we8[0]
    b0c = bb8 @ we8[0] + be8[0]
    h8, ps8, pd8, t8 = _embedproj_t0(nf8, wat_k, ba8, ws0c, wd0c, bs0, bd0,
                                     ef8, w0c, b0c)

    for l in range(L):
        last = l == L - 1
        outs = _edge_stage(ei2,
                           ps8.reshape(N, EDGE_DIM),
                           pd8.reshape(N, EDGE_DIM),
                           t8.reshape(E, EDGE_DIM),
                           write_eout=not last)
        if last:
            (aggp,) = outs
            h8 = _node_update(h8, aggp.reshape(_NC, N8, 8 * EDGE_DIM),
                              wh_k[l], wa_k[l], bn8[l])
        else:
            e_new, aggp = outs
            h8, ps8, pd8, t8 = _updproj_tmsg(
                h8, aggp.reshape(_NC, N8, 8 * EDGE_DIM),
                wh_k[l], wa_k[l], bn8[l],
                ws_k[l + 1], wd_k[l + 1],
                e_new.reshape(E8, 8 * EDGE_DIM), we8[l + 1], be8[l + 1])

    gi_f = graph_index.astype(jnp.float32).reshape(N, 1)
    return _readout(gi_f, h8.reshape(N, NODE_DIM), W_graph, b_graph, W_out)


# speedup vs baseline: 1.0814x; 1.0004x over previous
import jax
import jax.numpy as jnp
from jax import lax
from jax.experimental import pallas as pl
from jax.experimental.pallas import tpu as pltpu
from jax.experimental.pallas import tpu_sc as plsc

N = 10000
E = 320000
G = 128
NODE_DIM = 128
EDGE_DIM = 16
L = 3

_NC = 2
_NS = 16
_NW = _NC * _NS
CH = 1000
NCHUNK = E // CH
NJ = (NCHUNK + _NW - 1) // _NW
ROWS_PER_SUB = N // _NS
N8 = N // 8
E8 = E // 8


_BE = 2000
_NEB = E8 // _BE


def _embedproj_t0_body(nf8_ref, wat_ref, ba_ref, ws_ref, wd_ref, bs_ref,
                       bd_ref, ef_ref, w0_ref, b0_ref,
                       h8_ref, ps_ref, pd_ref, t_ref):
    i = pl.program_id(0)

    @pl.when(i == 0)
    def _node():
        nf8 = nf8_ref[...]
        h8_ref[...] = jnp.dot(nf8, wat_ref[...],
                              preferred_element_type=jnp.float32) + ba_ref[...]
        ps_ref[...] = jnp.dot(nf8, ws_ref[...],
                              preferred_element_type=jnp.float32) + bs_ref[...]
        pd_ref[...] = jnp.dot(nf8, wd_ref[...],
                              preferred_element_type=jnp.float32) + bd_ref[...]

    @pl.when(i > 0)
    def _edge():
        t_ref[...] = jnp.dot(ef_ref[...], w0_ref[...],
                             preferred_element_type=jnp.float32) + b0_ref[...]


def _embedproj_t0(nf8, wat_k, ba8, ws_c, wd_c, bs0, bd0, ef8, w0c, b0c):
    zz = lambda i: (0, 0)
    eb = lambda i: (jnp.maximum(i - 1, 0), 0)
    return pl.pallas_call(
        _embedproj_t0_body,
        grid=(1 + _NEB,),
        in_specs=[
            pl.BlockSpec((N8, 8 * NODE_DIM), zz),
            pl.BlockSpec((8 * NODE_DIM, 8 * NODE_DIM), zz),
            pl.BlockSpec((1, 8 * NODE_DIM), zz),
            pl.BlockSpec((8 * NODE_DIM, 8 * EDGE_DIM), zz),
            pl.BlockSpec((8 * NODE_DIM, 8 * EDGE_DIM), zz),
            pl.BlockSpec((1, 8 * EDGE_DIM), zz),
            pl.BlockSpec((1, 8 * EDGE_DIM), zz),
            pl.BlockSpec((_BE, 8 * EDGE_DIM), eb),
            pl.BlockSpec((8 * EDGE_DIM, 8 * EDGE_DIM), zz),
            pl.BlockSpec((1, 8 * EDGE_DIM), zz),
        ],
        out_specs=[
            pl.BlockSpec((N8, 8 * NODE_DIM), zz),
            pl.BlockSpec((N8, 8 * EDGE_DIM), zz),
            pl.BlockSpec((N8, 8 * EDGE_DIM), zz),
            pl.BlockSpec((_BE, 8 * EDGE_DIM), eb),
        ],
        out_shape=[
            jax.ShapeDtypeStruct((N8, 8 * NODE_DIM), jnp.float32),
            jax.ShapeDtypeStruct((N8, 8 * EDGE_DIM), jnp.float32),
            jax.ShapeDtypeStruct((N8, 8 * EDGE_DIM), jnp.float32),
            jax.ShapeDtypeStruct((E8, 8 * EDGE_DIM), jnp.float32),
        ],
    )(nf8, wat_k, ba8.reshape(1, -1), ws_c, wd_c,
      bs0.reshape(1, -1), bd0.reshape(1, -1), ef8, w0c, b0c.reshape(1, -1))


def _updproj_tmsg_body(h8_ref, agg8_ref, wh_ref, wa_ref, b_ref, ws_ref,
                       wd_ref, e8_ref, we_ref, be_ref,
                       o_ref, ps_ref, pd_ref, t_ref):
    i = pl.program_id(0)

    @pl.when(i == 0)
    def _node():
        a = agg8_ref[0] + agg8_ref[1]
        acc = jnp.dot(h8_ref[...], wh_ref[...],
                      preferred_element_type=jnp.float32)
        acc = acc + jnp.dot(a, wa_ref[...], preferred_element_type=jnp.float32)
        h8 = jnp.maximum(acc + b_ref[...], 0.0)
        o_ref[...] = h8
        ps_ref[...] = jnp.dot(h8, ws_ref[...],
                              preferred_element_type=jnp.float32)
        pd_ref[...] = jnp.dot(h8, wd_ref[...],
                              preferred_element_type=jnp.float32)

    @pl.when(i > 0)
    def _edge():
        t_ref[...] = jnp.dot(e8_ref[...], we_ref[...],
                             preferred_element_type=jnp.float32) + be_ref[...]


def _updproj_tmsg(h8, agg8, wh_k, wa_k, b8, ws_k, wd_k, e8, we_k, be8):
    zz = lambda i: (0, 0)
    eb = lambda i: (jnp.maximum(i - 1, 0), 0)
    return pl.pallas_call(
        _updproj_tmsg_body,
        grid=(1 + _NEB,),
        in_specs=[
            pl.BlockSpec((N8, 8 * NODE_DIM), zz),
            pl.BlockSpec((2, N8, 8 * EDGE_DIM), lambda i: (0, 0, 0)),
            pl.BlockSpec((8 * NODE_DIM, 8 * NODE_DIM), zz),
            pl.BlockSpec((8 * EDGE_DIM, 8 * NODE_DIM), zz),
            pl.BlockSpec((1, 8 * NODE_DIM), zz),
            pl.BlockSpec((8 * NODE_DIM, 8 * EDGE_DIM), zz),
            pl.BlockSpec((8 * NODE_DIM, 8 * EDGE_DIM), zz),
            pl.BlockSpec((_BE, 8 * EDGE_DIM), eb),
            pl.BlockSpec((8 * EDGE_DIM, 8 * EDGE_DIM), zz),
            pl.BlockSpec((1, 8 * EDGE_DIM), zz),
        ],
        out_specs=[
            pl.BlockSpec((N8, 8 * NODE_DIM), zz),
            pl.BlockSpec((N8, 8 * EDGE_DIM), zz),
            pl.BlockSpec((N8, 8 * EDGE_DIM), zz),
            pl.BlockSpec((_BE, 8 * EDGE_DIM), eb),
        ],
        out_shape=[
            jax.ShapeDtypeStruct((N8, 8 * NODE_DIM), jnp.float32),
            jax.ShapeDtypeStruct((N8, 8 * EDGE_DIM), jnp.float32),
            jax.ShapeDtypeStruct((N8, 8 * EDGE_DIM), jnp.float32),
            jax.ShapeDtypeStruct((E8, 8 * EDGE_DIM), jnp.float32),
        ],
    )(h8, agg8, wh_k, wa_k, b8.reshape(1, -1), ws_k, wd_k,
      e8, we_k, be8.reshape(1, -1))


def _node_update_body(h8_ref, agg8_ref, wh_ref, wa_ref, b_ref, o_ref):
    a = agg8_ref[0] + agg8_ref[1]
    acc = jnp.dot(h8_ref[...], wh_ref[...], preferred_element_type=jnp.float32)
    acc = acc + jnp.dot(a, wa_ref[...], preferred_element_type=jnp.float32)
    o_ref[...] = jnp.maximum(acc + b_ref[...], 0.0)


def _node_update(h8, agg8, wh_k, wa_k, b8):
    return pl.pallas_call(
        _node_update_body,
        grid=(1,),
        in_specs=[
            pl.BlockSpec((N8, 8 * NODE_DIM), lambda i: (0, 0)),
            pl.BlockSpec((2, N8, 8 * EDGE_DIM), lambda i: (0, 0, 0)),
            pl.BlockSpec((8 * NODE_DIM, 8 * NODE_DIM), lambda i: (0, 0)),
            pl.BlockSpec((8 * EDGE_DIM, 8 * NODE_DIM), lambda i: (0, 0)),
            pl.BlockSpec((1, 8 * NODE_DIM), lambda i: (0, 0)),
        ],
        out_specs=pl.BlockSpec((N8, 8 * NODE_DIM), lambda i: (0, 0)),
        out_shape=jax.ShapeDtypeStruct((N8, 8 * NODE_DIM), jnp.float32),
    )(h8, agg8, wh_k, wa_k, b8.reshape(1, -1))


def _readout_body(gi_ref, h_ref, wg_ref, bg_ref, wo_ref, o_ref, pooled_ref):
    i = pl.program_id(0)
    ng = pl.num_programs(0)

    @pl.when(i == 0)
    def _init():
        pooled_ref[...] = jnp.zeros_like(pooled_ref)

    gi = gi_ref[...]
    iota = lax.broadcasted_iota(jnp.int32, (gi.shape[0], G), 1).astype(jnp.float32)
    onehot = (gi == iota).astype(jnp.float32)
    contrib = lax.dot_general(onehot, h_ref[...],
                              (((0,), (0,)), ((), ())),
                              preferred_element_type=jnp.float32)
    pooled_ref[...] += contrib

    @pl.when(i == ng - 1)
    def _fin():
        g = jnp.maximum(
            jnp.dot(pooled_ref[...], wg_ref[...],
                    preferred_element_type=jnp.float32) + bg_ref[...], 0.0)
        o_ref[...] = jnp.dot(g, wo_ref[...],
                             preferred_element_type=jnp.float32)


def _readout(gi_f, h, wg, bg, wo, block_rows=1000):
    grid = N // block_rows
    return pl.pallas_call(
        _readout_body,
        grid=(grid,),
        in_specs=[
            pl.BlockSpec((block_rows, 1), lambda i: (i, 0)),
            pl.BlockSpec((block_rows, NODE_DIM), lambda i: (i, 0)),
            pl.BlockSpec((NODE_DIM, G), lambda i: (0, 0)),
            pl.BlockSpec((1, G), lambda i: (0, 0)),
            pl.BlockSpec((G, 256), lambda i: (0, 0)),
        ],
        out_specs=pl.BlockSpec((G, 256), lambda i: (0, 0)),
        out_shape=jax.ShapeDtypeStruct((G, 256), jnp.float32),
        scratch_shapes=[pltpu.VMEM((G, G), jnp.float32)],
    )(gi_f, h, wg, bg.reshape(1, -1), wo)


def _make_edge_body(write_eout):
    def body(ei_h, ps_h, pd_h, t_h, *rest):
        if write_eout:
            eout_h, agg_h = rest[0], rest[1]
            scratch = rest[2:]
        else:
            agg_h = rest[0]
            scratch = rest[1:]
        (sidxA, didxA, psvA, pdvA, tvA,
         sidxB, didxB, psvB, pdvB, tvB,
         obuf, agg_sh, semA, semB, wsemA, wsemB) = scratch
        c = lax.axis_index("c")
        s = lax.axis_index("s")
        wid = s * _NC + c
        CR = CH // 8

        def drain_w(j, tv, wsem):
            if not write_eout:
                return
            cid = j * _NW + wid

            @pl.when((j >= 0) & (cid < NCHUNK))
            def _():
                pltpu.make_async_copy(tv, eout_h.at[pl.ds(0, CH), :],
                                      wsem).wait()

        def fire(j, sidx, didx, psv, pdv, tv, sem, wsem):
            cid = j * _NW + wid
            drain_w(j - 2, tv, wsem)

            @pl.when(cid < NCHUNK)
            def _():
                pltpu.sync_copy(ei_h.at[0, pl.ds(cid * CH, CH)], sidx)
                pltpu.sync_copy(ei_h.at[1, pl.ds(cid * CH, CH)], didx)
                pltpu.async_copy(ps_h.at[sidx], psv, sem)
                pltpu.async_copy(pd_h.at[didx], pdv, sem)
                pltpu.async_copy(t_h.at[pl.ds(cid * CH, CH), :], tv, sem)

        def drain(j, psv, pdv, tv, sem):
            cid = j * _NW + wid

            @pl.when(cid < NCHUNK)
            def _():
                pltpu.make_async_copy(ps_h.at[pl.ds(0, CH), :],
                                      psv, sem).wait()
                pltpu.make_async_copy(ps_h.at[pl.ds(0, CH), :],
                                      pdv, sem).wait()
                pltpu.make_async_copy(t_h.at[pl.ds(0, CH), :], tv, sem).wait()

        def process(j, didx, psv, pdv, tv, wsem):
            cid = j * _NW + wid

            @pl.when(cid < NCHUNK)
            def _():
                def _row(i, carry):
                    tv[i, :] = jnp.maximum(psv[i, :] + pdv[i, :] + tv[i, :],
                                           0.0)
                    return carry
                lax.fori_loop(0, CH, _row, 0)
                if write_eout:
                    pltpu.async_copy(tv, eout_h.at[pl.ds(cid * CH, CH), :],
                                     wsem)
                pltpu.sync_copy(tv, agg_sh.at[didx], add=True)

        fire(0, sidxA, didxA, psvA, pdvA, tvA, semA, wsemA)

        def _zrow(i, carry):
            obuf[i, :] = jnp.zeros((16,), jnp.float32)
            return carry
        lax.fori_loop(0, ROWS_PER_SUB, _zrow, 0)
        pltpu.sync_copy(obuf,
                        agg_sh.at[pl.ds(s * ROWS_PER_SUB, ROWS_PER_SUB), :])
        plsc.subcore_barrier()

        def loop(j2, carry):
            jA = 2 * j2
            jB = jA + 1
            fire(jB, sidxB, didxB, psvB, pdvB, tvB, semB, wsemB)
            drain(jA, psvA, pdvA, tvA, semA)
            process(jA, didxA, psvA, pdvA, tvA, wsemA)
            fire(jA + 2, sidxA, didxA, psvA, pdvA, tvA, semA, wsemA)
            drain(jB, psvB, pdvB, tvB, semB)
            process(jB, didxB, psvB, pdvB, tvB, wsemB)
            return carry
        lax.fori_loop(0, NJ // 2, loop, 0)

        drain_w(NJ - 1, tvB, wsemB)

        plsc.subcore_barrier()
        pltpu.sync_copy(agg_sh.at[pl.ds(s * ROWS_PER_SUB, ROWS_PER_SUB), :],
                        obuf)
        pltpu.sync_copy(obuf,
                        agg_h.at[c, pl.ds(s * ROWS_PER_SUB, ROWS_PER_SUB), :])
    return body


def _edge_stage(ei2, ps, pd, t, write_eout=True):
    mesh = plsc.VectorSubcoreMesh(core_axis_name="c", subcore_axis_name="s",
                                  num_cores=_NC, num_subcores=_NS)
    agg_ty = jax.ShapeDtypeStruct((_NC, N, EDGE_DIM), jnp.float32)
    if write_eout:
        out_type = (jax.ShapeDtypeStruct((E, EDGE_DIM), jnp.float32),
                    agg_ty)
    else:
        out_type = (agg_ty,)
    f = pl.kernel(
        _make_edge_body(write_eout),
        out_type=out_type,
        mesh=mesh,
        compiler_params=pltpu.CompilerParams(use_tc_tiling_on_sc=False),
        scratch_types=[
            pltpu.VMEM((CH,), jnp.int32),
            pltpu.VMEM((CH,), jnp.int32),
            pltpu.VMEM((CH, EDGE_DIM), jnp.float32),
            pltpu.VMEM((CH, EDGE_DIM), jnp.float32),
            pltpu.VMEM((CH, EDGE_DIM), jnp.float32),
            pltpu.VMEM((CH,), jnp.int32),
            pltpu.VMEM((CH,), jnp.int32),
            pltpu.VMEM((CH, EDGE_DIM), jnp.float32),
            pltpu.VMEM((CH, EDGE_DIM), jnp.float32),
            pltpu.VMEM((CH, EDGE_DIM), jnp.float32),
            pltpu.VMEM((ROWS_PER_SUB, EDGE_DIM), jnp.float32),
            pltpu.VMEM_SHARED((N, EDGE_DIM), jnp.float32),
            pltpu.SemaphoreType.DMA,
            pltpu.SemaphoreType.DMA,
            pltpu.SemaphoreType.DMA,
            pltpu.SemaphoreType.DMA,
        ],
    )
    return f(ei2, ps, pd, t)


def kernel(node_features, edge_features, edge_index, graph_index,
           W_atom, b_atom, W_bond, b_bond, W_edge, b_edge,
           W_node, b_node, W_graph, b_graph, W_out):
    ei2 = edge_index.astype(jnp.int32)
    eye8 = jnp.eye(8, dtype=jnp.float32)

    def ekron(w):
        return jnp.kron(eye8, w)

    ws_k = [ekron(W_edge[l, :NODE_DIM, :]) for l in range(L)]
    wd_k = [ekron(W_edge[l, NODE_DIM:2 * NODE_DIM, :]) for l in range(L)]
    we8 = [ekron(W_edge[l, 2 * NODE_DIM:, :]) for l in range(L)]
    be8 = [jnp.tile(b_edge[l], 8) for l in range(L)]
    wh_k = [ekron(W_node[l, :NODE_DIM, :]) for l in range(L)]
    wa_k = [ekron(W_node[l, NODE_DIM:, :]) for l in range(L)]
    bn8 = [jnp.tile(b_node[l], 8) for l in range(L)]

    nf8 = node_features.reshape(N8, 8 * NODE_DIM)
    wat_k = ekron(W_atom)
    ba8 = jnp.tile(b_atom, 8)
    ws0c = ekron(W_atom @ W_edge[0, :NODE_DIM, :])
    wd0c = ekron(W_atom @ W_edge[0, NODE_DIM:2 * NODE_DIM, :])
    bs0 = jnp.tile(b_atom @ W_edge[0, :NODE_DIM, :], 8)
    bd0 = jnp.tile(b_atom @ W_edge[0, NODE_DIM:2 * NODE_DIM, :], 8)
    ef8 = edge_features.reshape(E8, 8 * EDGE_DIM)
    wb8 = ekron(W_bond)
    bb8 = jnp.tile(b_bond, 8)
    w0c = wb8 @ we8[0]
    b0c = bb8 @ we8[0] + be8[0]
    h8, ps8, pd8, t8 = _embedproj_t0(nf8, wat_k, ba8, ws0c, wd0c, bs0, bd0,
                                     ef8, w0c, b0c)

    for l in range(L):
        last = l == L - 1
        outs = _edge_stage(ei2,
                           ps8.reshape(N, EDGE_DIM),
                           pd8.reshape(N, EDGE_DIM),
                           t8.reshape(E, EDGE_DIM),
                           write_eout=not last)
        if last:
            (aggp,) = outs
            h8 = _node_update(h8, aggp.reshape(_NC, N8, 8 * EDGE_DIM),
                              wh_k[l], wa_k[l], bn8[l])
        else:
            e_new, aggp = outs
            h8, ps8, pd8, t8 = _updproj_tmsg(
                h8, aggp.reshape(_NC, N8, 8 * EDGE_DIM),
                wh_k[l], wa_k[l], bn8[l],
                ws_k[l + 1], wd_k[l + 1],
                e_new.reshape(E8, 8 * EDGE_DIM), we8[l + 1], be8[l + 1])

    gi_f = graph_index.astype(jnp.float32).reshape(N, 1)
    return _readout(gi_f, h8.reshape(N, NODE_DIM), W_graph, b_graph, W_out)
